# baseline (speedup 1.0000x reference)
import jax
import jax.numpy as jnp
from jax import lax
from jax.experimental import pallas as pl
from jax.experimental.pallas import tpu as pltpu
from jax.experimental.pallas import tpu_sc as plsc

NC, NS, L = 2, 16, 16
NW = NC * NS
ROWS, COLS = 128, 32768
RPW = ROWS // NW
K = 8
GROUP = 32
GW = GROUP * L
NGRP = COLS // GW
CAP = 2048
QCAP = 16
NEG = -100000.0
IMAX = 2**31 - 1


def _body(scores_hbm, kofs_hbm, masked_hbm, vals_hbm, idx_hbm,
          row0_v, row1_v, neg_v_buf, cvals_v, cidx_v, pstage_v,
          kofs_v, pack_f, pack_i, gsm_s,
          sin0, sin1, sneg0, sneg1, sneg2, sneg3, sscat, spack):
    rowbufs = [row0_v, row1_v]
    sneg = [sneg0, sneg1, sneg2, sneg3]
    wid = lax.axis_index("s") * NC + lax.axis_index("c")
    pltpu.sync_copy(kofs_hbm, kofs_v)
    kofs = jnp.max(kofs_v[...])
    lanes = lax.broadcasted_iota(jnp.int32, (L,), 0)
    ninf = jnp.float32(-jnp.inf)
    ninf_v = jnp.full((L,), ninf, jnp.float32)
    neg_vec = jnp.full((L,), NEG, jnp.float32)
    row_base = wid * RPW

    in_h = [None] * RPW
    in_h[0] = pltpu.async_copy(scores_hbm.at[row_base], rowbufs[0], sin0)
    in_h_sem = [sin0, sin1]

    def negfill(j, _c):
        for u in range(8):
            neg_v_buf[pl.ds((j * 8 + u) * L, L)] = neg_vec
        return 0
    lax.fori_loop(0, COLS // (8 * L), negfill, 0)
    neg_h = [pltpu.async_copy(neg_v_buf, masked_hbm.at[row_base + r], sneg[r])
             for r in range(RPW)]

    tvpack = ninf_v
    tipack = jnp.zeros((L,), jnp.int32)
    q_hist = [None] * RPW

    for r in range(RPW):
        buf = rowbufs[r % 2]
        row = row_base + r
        in_h[r].wait()
        if r + 1 < RPW:
            in_h[r + 1] = pltpu.async_copy(
                scores_hbm.at[row + 1], rowbufs[(r + 1) % 2],
                in_h_sem[(r + 1) % 2])

        def grp1(g, lm, buf=buf):
            acc = [ninf_v] * 8
            for j in range(GROUP):
                acc[j % 8] = jnp.maximum(
                    acc[j % 8], buf[pl.ds((g * GROUP + j) * L, L)])
            a0 = jnp.maximum(jnp.maximum(acc[0], acc[1]),
                             jnp.maximum(acc[2], acc[3]))
            a1 = jnp.maximum(jnp.maximum(acc[4], acc[5]),
                             jnp.maximum(acc[6], acc[7]))
            gm = jnp.maximum(a0, a1)
            gsm_s[g] = jnp.max(gm)
            return jnp.maximum(lm, gm)
        lm = lax.fori_loop(0, NGRP, grp1, ninf_v)

        t = ninf
        for _i in range(K):
            t = jnp.max(lm)
            lm = jnp.where(lm == t, ninf_v, lm)

        def grp2(g, off, buf=buf, t=t):
            def collect(off):
                for j in range(GROUP):
                    base = (g * GROUP + j) * L
                    v = buf[pl.ds(base, L)]
                    m = v >= t
                    cnt = jnp.sum(m.astype(jnp.int32))
                    o = jnp.minimum(off, CAP)
                    plsc.store_compressed(cvals_v.at[pl.ds(o, L)], v, mask=m)
                    plsc.store_compressed(cidx_v.at[pl.ds(o, L)],
                                          lanes + base, mask=m)
                    off = jnp.minimum(off + cnt, CAP)
                return off
            return lax.cond(gsm_s[g] >= t, collect, lambda o: o, off)
        used = lax.fori_loop(0, NGRP, grp2, jnp.int32(0))
        nv = (used + L - 1) // L
        cvals_v[pl.ds(used, L)] = ninf_v

        lane_base = (r % 2) * K

        def round_fn(i, carry, nv=nv, lane_base=lane_base):
            tv, ti, _v8 = carry

            def amax(jv, m):
                return jnp.maximum(m, cvals_v[pl.ds(jv * L, L)])
            mx = jnp.max(lax.fori_loop(0, nv, amax, ninf_v))

            def amin(jv, mi):
                cv = cvals_v[pl.ds(jv * L, L)]
                ci = cidx_v[pl.ds(jv * L, L)]
                return jnp.minimum(mi, jnp.where(cv == mx, ci, IMAX))
            mix = jnp.min(lax.fori_loop(0, nv, amin,
                                        jnp.full((L,), IMAX, jnp.int32)))

            def rem(jv, _c):
                cv = cvals_v[pl.ds(jv * L, L)]
                ci = cidx_v[pl.ds(jv * L, L)]
                cvals_v[pl.ds(jv * L, L)] = jnp.where(ci == mix, ninf_v, cv)
                return 0
            lax.fori_loop(0, nv, rem, 0)
            tv = jnp.where(lanes == lane_base + i, mx, tv)
            ti = jnp.where(lanes == lane_base + i, mix, ti)
            return tv, ti, mx
        tvpack, tipack, v8 = lax.fori_loop(
            0, K, round_fn, (tvpack, tipack, ninf))
        if r % 2 == 1:
            pack_f[pl.ds((r // 2) * L, L)] = tvpack + kofs
            pack_i[pl.ds((r // 2) * L, L)] = tipack
            tvpack = ninf_v
            tipack = jnp.zeros((L,), jnp.int32)

        neg_h[r].wait()
        if r >= 1:
            def drain(_j, _c):
                pltpu.make_async_copy(
                    scores_hbm.at[row_base].at[pl.ds(0, GW)],
                    pstage_v.at[pl.ds(0, GW)], sscat).wait()
                return 0
            lax.fori_loop(0, q_hist[r - 1], drain, 0)

        def patchgrp(g, q, buf=buf, row=row, v8=v8):
            def dopatch(q):
                q_c = jnp.minimum(q, QCAP - 1)
                for j in range(GROUP):
                    v = buf[pl.ds((g * GROUP + j) * L, L)]
                    pstage_v[pl.ds(q_c * GW + j * L, L)] = jnp.where(
                        v >= v8, v + kofs, neg_vec)

                @pl.when(q < QCAP)
                def _():
                    pltpu.async_copy(
                        pstage_v.at[pl.ds(q_c * GW, GW)],
                        masked_hbm.at[row].at[pl.ds(g * GW, GW)], sscat)
                return q + 1
            return lax.cond(gsm_s[g] >= v8, dopatch, lambda q: q, q)
        q = lax.fori_loop(0, NGRP, patchgrp, jnp.int32(0))
        q_hist[r] = jnp.minimum(q, QCAP)

    def drain_last(_j, _c):
        pltpu.make_async_copy(
            scores_hbm.at[row_base].at[pl.ds(0, GW)],
            pstage_v.at[pl.ds(0, GW)], sscat).wait()
        return 0
    lax.fori_loop(0, q_hist[RPW - 1], drain_last, 0)

    pltpu.async_copy(pack_f, vals_hbm.at[pl.ds(row_base * K, RPW * K)],
                     spack).wait()
    pltpu.async_copy(pack_i, idx_hbm.at[pl.ds(row_base * K, RPW * K)],
                     spack).wait()


def kernel(scores, k):
    kofs = jnp.full((L,), 1.0, jnp.float32) * (
        jnp.asarray(k, jnp.int32) - K).astype(jnp.float32)
    mesh = plsc.VectorSubcoreMesh(core_axis_name="c", subcore_axis_name="s",
                                  num_cores=NC, num_subcores=NS)
    f = pl.kernel(
        _body,
        out_type=[
            jax.ShapeDtypeStruct((ROWS, COLS), jnp.float32),
            jax.ShapeDtypeStruct((ROWS * K,), jnp.float32),
            jax.ShapeDtypeStruct((ROWS * K,), jnp.int32),
        ],
        mesh=mesh,
        compiler_params=pltpu.CompilerParams(needs_layout_passes=False),
        scratch_types=[
            pltpu.VMEM((COLS,), jnp.float32),
            pltpu.VMEM((COLS,), jnp.float32),
            pltpu.VMEM((COLS,), jnp.float32),
            pltpu.VMEM((CAP + L,), jnp.float32),
            pltpu.VMEM((CAP + L,), jnp.int32),
            pltpu.VMEM((QCAP * GW,), jnp.float32),
            pltpu.VMEM((L,), jnp.float32),
            pltpu.VMEM((RPW * K,), jnp.float32),
            pltpu.VMEM((RPW * K,), jnp.int32),
            pltpu.SMEM((NGRP,), jnp.float32),
            pltpu.SemaphoreType.DMA,
            pltpu.SemaphoreType.DMA,
            pltpu.SemaphoreType.DMA,
            pltpu.SemaphoreType.DMA,
            pltpu.SemaphoreType.DMA,
            pltpu.SemaphoreType.DMA,
            pltpu.SemaphoreType.DMA,
            pltpu.SemaphoreType.DMA,
        ],
    )
    masked, vals, idx = f(scores, kofs)
    return masked, vals.reshape(ROWS, K), idx.reshape(ROWS, K)


# speedup vs baseline: 1.1526x; 1.1526x over previous
import jax
import jax.numpy as jnp
from jax import lax
from jax.experimental import pallas as pl
from jax.experimental.pallas import tpu as pltpu
from jax.experimental.pallas import tpu_sc as plsc

NC, NS, L = 2, 16, 16
NW = NC * NS
ROWS, COLS = 128, 32768
RPW = ROWS // NW
K = 8
GROUP = 16
GW = GROUP * L
NGRP = COLS // GW
CAP = 2048
QCAP = 32
NEG = -100000.0
IMAX = 2**31 - 1


def _body(scores_hbm, kofs_hbm, masked_hbm, vals_hbm, idx_hbm,
          row0_v, row1_v, neg_v_buf, cvals_v, cidx_v, pstage_v,
          kofs_v, pack_f, pack_i, gsm_s,
          sin0, sin1, sneg0, sneg1, sneg2, sneg3, sscat, spack):
    rowbufs = [row0_v, row1_v]
    sneg = [sneg0, sneg1, sneg2, sneg3]
    wid = lax.axis_index("s") * NC + lax.axis_index("c")
    pltpu.sync_copy(kofs_hbm, kofs_v)
    kofs = jnp.max(kofs_v[...])
    lanes = lax.broadcasted_iota(jnp.int32, (L,), 0)
    ninf = jnp.float32(-jnp.inf)
    ninf_v = jnp.full((L,), ninf, jnp.float32)
    neg_vec = jnp.full((L,), NEG, jnp.float32)
    row_base = wid * RPW

    in_h = [None] * RPW
    in_h[0] = pltpu.async_copy(scores_hbm.at[row_base], rowbufs[0], sin0)
    in_h_sem = [sin0, sin1]

    def negfill(j, _c):
        for u in range(8):
            neg_v_buf[pl.ds((j * 8 + u) * L, L)] = neg_vec
        return 0
    lax.fori_loop(0, COLS // (8 * L), negfill, 0)
    neg_h = [pltpu.async_copy(neg_v_buf, masked_hbm.at[row_base + r], sneg[r])
             for r in range(RPW)]

    tvpack = ninf_v
    tipack = jnp.zeros((L,), jnp.int32)
    q_hist = [None] * RPW

    for r in range(RPW):
        buf = rowbufs[r % 2]
        row = row_base + r
        in_h[r].wait()
        if r + 1 < RPW:
            in_h[r + 1] = pltpu.async_copy(
                scores_hbm.at[row + 1], rowbufs[(r + 1) % 2],
                in_h_sem[(r + 1) % 2])

        def grp1(g, lm, buf=buf):
            acc = [ninf_v] * 4
            for j in range(GROUP):
                acc[j % 4] = jnp.maximum(
                    acc[j % 4], buf[pl.ds((g * GROUP + j) * L, L)])
            gm = jnp.maximum(jnp.maximum(acc[0], acc[1]),
                             jnp.maximum(acc[2], acc[3]))
            gsm_s[g] = jnp.max(gm)
            return jnp.maximum(lm, gm)
        lm = lax.fori_loop(0, NGRP, grp1, ninf_v)

        t = ninf
        for _i in range(K):
            t = jnp.max(lm)
            lm = jnp.where(lm == t, ninf_v, lm)

        def grp2(g, off, buf=buf, t=t):
            def collect(off):
                for j in range(GROUP):
                    base = (g * GROUP + j) * L
                    v = buf[pl.ds(base, L)]
                    m = v >= t
                    cnt = jnp.sum(m.astype(jnp.int32))
                    o = jnp.minimum(off, CAP)
                    plsc.store_compressed(cvals_v.at[pl.ds(o, L)], v, mask=m)
                    plsc.store_compressed(cidx_v.at[pl.ds(o, L)],
                                          lanes + base, mask=m)
                    off = jnp.minimum(off + cnt, CAP)
                return off
            return lax.cond(gsm_s[g] >= t, collect, lambda o: o, off)
        used = lax.fori_loop(0, NGRP, grp2, jnp.int32(0))
        nv = (used + L - 1) // L
        cvals_v[pl.ds(used, L)] = ninf_v

        lane_base = (r % 2) * K

        def round_fn(i, carry, nv=nv, lane_base=lane_base):
            tv, ti, _v8 = carry

            def amax(jv, m):
                return jnp.maximum(m, cvals_v[pl.ds(jv * L, L)])
            mx = jnp.max(lax.fori_loop(0, nv, amax, ninf_v))

            def amin(jv, mi):
                cv = cvals_v[pl.ds(jv * L, L)]
                ci = cidx_v[pl.ds(jv * L, L)]
                return jnp.minimum(mi, jnp.where(cv == mx, ci, IMAX))
            mix = jnp.min(lax.fori_loop(0, nv, amin,
                                        jnp.full((L,), IMAX, jnp.int32)))

            def rem(jv, _c):
                cv = cvals_v[pl.ds(jv * L, L)]
                ci = cidx_v[pl.ds(jv * L, L)]
                cvals_v[pl.ds(jv * L, L)] = jnp.where(ci == mix, ninf_v, cv)
                return 0
            lax.fori_loop(0, nv, rem, 0)
            tv = jnp.where(lanes == lane_base + i, mx, tv)
            ti = jnp.where(lanes == lane_base + i, mix, ti)
            return tv, ti, mx
        tvpack, tipack, v8 = lax.fori_loop(
            0, K, round_fn, (tvpack, tipack, ninf))
        if r % 2 == 1:
            pack_f[pl.ds((r // 2) * L, L)] = tvpack + kofs
            pack_i[pl.ds((r // 2) * L, L)] = tipack
            tvpack = ninf_v
            tipack = jnp.zeros((L,), jnp.int32)

        neg_h[r].wait()
        if r >= 1:
            def drain(_j, _c):
                pltpu.make_async_copy(
                    scores_hbm.at[row_base].at[pl.ds(0, GW)],
                    pstage_v.at[pl.ds(0, GW)], sscat).wait()
                return 0
            lax.fori_loop(0, q_hist[r - 1], drain, 0)

        def patchgrp(g, q, buf=buf, row=row, v8=v8):
            def dopatch(q):
                q_c = jnp.minimum(q, QCAP - 1)
                for j in range(GROUP):
                    v = buf[pl.ds((g * GROUP + j) * L, L)]
                    pstage_v[pl.ds(q_c * GW + j * L, L)] = jnp.where(
                        v >= v8, v + kofs, neg_vec)

                @pl.when(q < QCAP)
                def _():
                    pltpu.async_copy(
                        pstage_v.at[pl.ds(q_c * GW, GW)],
                        masked_hbm.at[row].at[pl.ds(g * GW, GW)], sscat)
                return q + 1
            return lax.cond(gsm_s[g] >= v8, dopatch, lambda q: q, q)
        q = lax.fori_loop(0, NGRP, patchgrp, jnp.int32(0))
        q_hist[r] = jnp.minimum(q, QCAP)

    def drain_last(_j, _c):
        pltpu.make_async_copy(
            scores_hbm.at[row_base].at[pl.ds(0, GW)],
            pstage_v.at[pl.ds(0, GW)], sscat).wait()
        return 0
    lax.fori_loop(0, q_hist[RPW - 1], drain_last, 0)

    pltpu.async_copy(pack_f, vals_hbm.at[pl.ds(row_base * K, RPW * K)],
                     spack).wait()
    pltpu.async_copy(pack_i, idx_hbm.at[pl.ds(row_base * K, RPW * K)],
                     spack).wait()


def kernel(scores, k):
    kofs = jnp.full((L,), 1.0, jnp.float32) * (
        jnp.asarray(k, jnp.int32) - K).astype(jnp.float32)
    mesh = plsc.VectorSubcoreMesh(core_axis_name="c", subcore_axis_name="s",
                                  num_cores=NC, num_subcores=NS)
    f = pl.kernel(
        _body,
        out_type=[
            jax.ShapeDtypeStruct((ROWS, COLS), jnp.float32),
            jax.ShapeDtypeStruct((ROWS * K,), jnp.float32),
            jax.ShapeDtypeStruct((ROWS * K,), jnp.int32),
        ],
        mesh=mesh,
        compiler_params=pltpu.CompilerParams(needs_layout_passes=False),
        scratch_types=[
            pltpu.VMEM((COLS,), jnp.float32),
            pltpu.VMEM((COLS,), jnp.float32),
            pltpu.VMEM((COLS,), jnp.float32),
            pltpu.VMEM((CAP + L,), jnp.float32),
            pltpu.VMEM((CAP + L,), jnp.int32),
            pltpu.VMEM((QCAP * GW,), jnp.float32),
            pltpu.VMEM((L,), jnp.float32),
            pltpu.VMEM((RPW * K,), jnp.float32),
            pltpu.VMEM((RPW * K,), jnp.int32),
            pltpu.SMEM((NGRP,), jnp.float32),
            pltpu.SemaphoreType.DMA,
            pltpu.SemaphoreType.DMA,
            pltpu.SemaphoreType.DMA,
            pltpu.SemaphoreType.DMA,
            pltpu.SemaphoreType.DMA,
            pltpu.SemaphoreType.DMA,
            pltpu.SemaphoreType.DMA,
            pltpu.SemaphoreType.DMA,
        ],
    )
    masked, vals, idx = f(scores, kofs)
    return masked, vals.reshape(ROWS, K), idx.reshape(ROWS, K)
